# Initial kernel scaffold; baseline (speedup 1.0000x reference)
import jax
import jax.numpy as jnp
from jax.experimental import pallas as pl


def kernel(x, edge_index, W_enc, b_enc, W_self0, W_nei0, b0, W_self1, W_nei1, b1, W_head, b_head, num_seed):
    raise NotImplementedError("write your pallas kernel here")


# speedup vs baseline: 11.0997x; 11.0997x over previous
import functools

import jax
import jax.numpy as jnp
from jax import lax
from jax.experimental import pallas as pl
from jax.experimental.pallas import tpu as pltpu
from jax.experimental.pallas import tpu_sc as plsc

NC = 2
NS = 16
NW = NC * NS
K = 125
SZ = 80


def _rup(a, m):
    return (a + m - 1) // m * m


def _make_seg_kernel(N, C, NP, RT, EPT, NCH):
    mesh = plsc.VectorSubcoreMesh(core_axis_name="c", subcore_axis_name="s")

    NPH = NCH // 2

    @functools.partial(
        pl.kernel,
        mesh=mesh,
        out_type=jax.ShapeDtypeStruct((NC, NP, C), jnp.float32),
        scratch_types=[
            pltpu.VMEM((NPH, K), jnp.int32),
            pltpu.VMEM((NPH, K), jnp.int32),
            pltpu.VMEM((K, C), jnp.float32),
            pltpu.VMEM((K, C), jnp.float32),
            pltpu.VMEM_SHARED((NP, C), jnp.float32),
            pltpu.SemaphoreType.DMA,
            pltpu.SemaphoreType.DMA,
        ],
    )
    def seg(h_hbm, src_hbm, dst_hbm, z_hbm, out_hbm, srcs_v, dsts_v,
            rows0, rows1, acc, sem0, sem1):
        c = lax.axis_index("c")
        s = lax.axis_index("s")
        wid = s * NC + c
        pltpu.sync_copy(z_hbm, rows0.at[pl.ds(0, SZ)])
        for t in range(RT // SZ):
            pltpu.sync_copy(rows0.at[pl.ds(0, SZ)],
                            acc.at[pl.ds(s * RT + t * SZ, SZ)])
        plsc.subcore_barrier()

        def gath(chunk, buf, sem):
            return pltpu.make_async_copy(h_hbm.at[srcs_v.at[chunk]], buf, sem)

        def scat(chunk, buf):
            pltpu.sync_copy(buf, acc.at[dsts_v.at[chunk]], add=True)

        def body(i, carry):
            i0 = 2 * i
            gath(i0, rows0, sem0).wait()
            gath(i0 + 1, rows1, sem1).start()
            scat(i0, rows0)
            gath(i0 + 1, rows1, sem1).wait()
            gath(i0 + 2, rows0, sem0).start()
            scat(i0 + 1, rows1)
            return carry

        P = (NPH - 1) // 2
        for ph in range(NCH // NPH):
            pltpu.sync_copy(src_hbm.at[wid, pl.ds(ph * NPH, NPH)], srcs_v)
            pltpu.sync_copy(dst_hbm.at[wid, pl.ds(ph * NPH, NPH)], dsts_v)
            gath(0, rows0, sem0).start()
            if P > 0:
                lax.fori_loop(0, P, body, 0)
            if NPH - 2 * P == 1:
                gath(NPH - 1, rows0, sem0).wait()
                scat(NPH - 1, rows0)
            else:
                gath(NPH - 2, rows0, sem0).wait()
                gath(NPH - 1, rows1, sem1).start()
                scat(NPH - 2, rows0)
                gath(NPH - 1, rows1, sem1).wait()
                scat(NPH - 1, rows1)
        plsc.subcore_barrier()
        for t in range(RT // SZ):
            pltpu.sync_copy(acc.at[pl.ds(s * RT + t * SZ, SZ)],
                            rows0.at[pl.ds(0, SZ)])
            pltpu.sync_copy(rows0.at[pl.ds(0, SZ)],
                            out_hbm.at[c, pl.ds(s * RT + t * SZ, SZ)])

    return seg


def _make_segq_kernel(N, NP, RT, EPT, NCH):
    mesh = plsc.VectorSubcoreMesh(core_axis_name="c", subcore_axis_name="s")

    @functools.partial(
        pl.kernel,
        mesh=mesh,
        out_type=jax.ShapeDtypeStruct((NC * NP,), jnp.float32),
        scratch_types=[
            pltpu.VMEM((NCH, K), jnp.int32),
            pltpu.VMEM((NCH, K), jnp.int32),
            pltpu.VMEM((K,), jnp.float32),
            pltpu.VMEM((K,), jnp.float32),
            pltpu.VMEM((SZ,), jnp.float32),
            pltpu.VMEM_SHARED((NP,), jnp.float32),
            pltpu.SemaphoreType.DMA,
            pltpu.SemaphoreType.DMA,
        ],
    )
    def segq(p_hbm, src_hbm, dst_hbm, z_hbm, out_hbm,
             srcs_v, dsts_v, val0, val1, stz, qacc, sem0, sem1):
        c = lax.axis_index("c")
        s = lax.axis_index("s")
        wid = s * NC + c
        pltpu.sync_copy(z_hbm, stz)
        for t in range(RT // SZ):
            pltpu.sync_copy(stz, qacc.at[pl.ds(s * RT + t * SZ, SZ)])
        pltpu.sync_copy(src_hbm.at[wid], srcs_v)
        pltpu.sync_copy(dst_hbm.at[wid], dsts_v)
        plsc.subcore_barrier()

        def gath(chunk, buf, sem):
            return pltpu.make_async_copy(p_hbm.at[srcs_v.at[chunk]], buf, sem)

        def scat(chunk, buf):
            pltpu.sync_copy(buf, qacc.at[dsts_v.at[chunk]], add=True)

        gath(0, val0, sem0).start()
        P = (NCH - 1) // 2

        def body(i, carry):
            i0 = 2 * i
            gath(i0, val0, sem0).wait()
            gath(i0 + 1, val1, sem1).start()
            scat(i0, val0)
            gath(i0 + 1, val1, sem1).wait()
            gath(i0 + 2, val0, sem0).start()
            scat(i0 + 1, val1)
            return carry

        if P > 0:
            lax.fori_loop(0, P, body, 0)
        if NCH - 2 * P == 1:
            gath(NCH - 1, val0, sem0).wait()
            scat(NCH - 1, val0)
        else:
            gath(NCH - 2, val0, sem0).wait()
            gath(NCH - 1, val1, sem1).start()
            scat(NCH - 2, val0)
            gath(NCH - 1, val1, sem1).wait()
            scat(NCH - 1, val1)
        plsc.subcore_barrier()
        for t in range(RT // SZ):
            pltpu.sync_copy(qacc.at[pl.ds(s * RT + t * SZ, SZ)], stz)
            pltpu.sync_copy(stz, out_hbm.at[pl.ds(c * NP + s * RT + t * SZ, SZ)])

    return segq


def _dot(a, b):
    return jnp.dot(a, b, preferred_element_type=jnp.float32,
                   precision=lax.Precision.HIGHEST)


def _enc_body(x_ref, w_ref, b_ref, o_ref):
    o_ref[...] = _dot(x_ref[...], w_ref[...]) + b_ref[...]


def _mid_body(h_ref, ma_ref, mb_ref, ws_ref, wn_ref, b_ref, wc_ref, o_ref):
    m = ma_ref[...] + mb_ref[...]
    h1 = _dot(h_ref[...], ws_ref[...]) + _dot(m, wn_ref[...])
    h1 = jnp.maximum(h1 + b_ref[...], 0.0)
    o_ref[...] = _dot(h1, wc_ref[...])


@jax.jit
def kernel(x, edge_index, W_enc, b_enc, W_self0, W_nei0, b0,
           W_self1, W_nei1, b1, W_head, b_head, num_seed):
    N, C = x.shape
    E = edge_index.shape[1]
    RT = _rup(-(-N // NS), SZ)
    NP = RT * NS
    EPAD = _rup(E, NW * K * 16)
    if EPAD != E and NP == N:
        RT += SZ
        NP = RT * NS
    EPT = EPAD // NW
    NCH = EPT // K

    src = edge_index[0]
    dst = edge_index[1]
    if EPAD != E:
        src = jnp.concatenate([src, jnp.zeros((EPAD - E,), jnp.int32)])
        dst = jnp.concatenate([dst, jnp.full((EPAD - E,), N, jnp.int32)])
    src = src.reshape(NW, NCH, K)
    dst = dst.reshape(NW, NCH, K)

    BR = 1000 if N % 1000 == 0 else 8
    NB = N // BR
    row_spec = pl.BlockSpec((BR, C), lambda i: (i, 0))
    w_spec = pl.BlockSpec((C, C), lambda i: (0, 0))
    b_spec = pl.BlockSpec((1, C), lambda i: (0, 0))

    h = pl.pallas_call(
        _enc_body,
        grid=(NB,),
        in_specs=[row_spec, w_spec, b_spec],
        out_specs=row_spec,
        out_shape=jax.ShapeDtypeStruct((N, C), jnp.float32),
    )(x, W_enc, b_enc.reshape(1, C))

    zrow = jnp.zeros((SZ, C), jnp.float32)
    seg = _make_seg_kernel(N, C, NP, RT, EPT, NCH)
    msg = seg(h, src, dst, zrow)

    wnh = W_nei1 @ W_head
    wsh = W_self1 @ W_head
    Wc = jnp.concatenate([wnh, wsh], axis=1)
    ps = pl.pallas_call(
        _mid_body,
        grid=(NB,),
        in_specs=[row_spec, row_spec, row_spec, w_spec, w_spec, b_spec,
                  pl.BlockSpec((C, 2), lambda i: (0, 0))],
        out_specs=pl.BlockSpec((BR, 2), lambda i: (i, 0)),
        out_shape=jax.ShapeDtypeStruct((N, 2), jnp.float32),
    )(h, msg[0, :N], msg[1, :N], W_self0, W_nei0, b0.reshape(1, C), Wc)

    p = ps[:, 0] + jnp.float32(0.0)
    s_full = ps[:, 1]
    zq = jnp.zeros((SZ,), jnp.float32)
    segq = _make_segq_kernel(N, NP, RT, EPT, NCH)
    q = segq(p, src, dst, zq).reshape(NC, NP)

    tot = s_full + q[0, :N] + q[1, :N]
    seed = lax.dynamic_slice(tot, (num_seed - 1024,), (1024,))
    return seed[:, None] + (b1 @ W_head)[None, :] + b_head[None, :]


# speedup vs baseline: 13.1251x; 1.1825x over previous
import functools

import jax
import jax.numpy as jnp
from jax import lax
from jax.experimental import pallas as pl
from jax.experimental.pallas import tpu as pltpu
from jax.experimental.pallas import tpu_sc as plsc

NC = 2
NS = 16
NW = NC * NS
K = 125
SZ = 80


def _rup(a, m):
    return (a + m - 1) // m * m


def _make_seg_kernel(N, C, NP, RT, EPT, NCH):
    mesh = plsc.VectorSubcoreMesh(core_axis_name="c", subcore_axis_name="s")

    NPH = NCH // 2

    @functools.partial(
        pl.kernel,
        mesh=mesh,
        out_type=jax.ShapeDtypeStruct((NC, NP, C), jnp.float32),
        scratch_types=[
            pltpu.VMEM((NPH, K), jnp.int32),
            pltpu.VMEM((NPH, K), jnp.int32),
            pltpu.VMEM((K, C), jnp.float32),
            pltpu.VMEM((K, C), jnp.float32),
            pltpu.VMEM_SHARED((NP, C), jnp.float32),
            pltpu.SemaphoreType.DMA,
            pltpu.SemaphoreType.DMA,
        ],
    )
    def seg(h_hbm, src_hbm, dst_hbm, z_hbm, out_hbm, srcs_v, dsts_v,
            rows0, rows1, acc, sem0, sem1):
        c = lax.axis_index("c")
        s = lax.axis_index("s")
        wid = s * NC + c
        pltpu.sync_copy(z_hbm, rows0.at[pl.ds(0, SZ)])
        for t in range(RT // SZ):
            pltpu.sync_copy(rows0.at[pl.ds(0, SZ)],
                            acc.at[pl.ds(s * RT + t * SZ, SZ)])
        plsc.subcore_barrier()

        def gath(chunk, buf, sem):
            return pltpu.make_async_copy(h_hbm.at[srcs_v.at[chunk]], buf, sem)

        def scat(chunk, buf):
            pltpu.sync_copy(buf, acc.at[dsts_v.at[chunk]], add=True)

        def body(i, carry):
            i0 = 2 * i
            gath(i0, rows0, sem0).wait()
            gath(i0 + 1, rows1, sem1).start()
            scat(i0, rows0)
            gath(i0 + 1, rows1, sem1).wait()
            gath(i0 + 2, rows0, sem0).start()
            scat(i0 + 1, rows1)
            return carry

        P = (NPH - 1) // 2
        for ph in range(NCH // NPH):
            pltpu.sync_copy(src_hbm.at[wid, pl.ds(ph * NPH, NPH)], srcs_v)
            pltpu.sync_copy(dst_hbm.at[wid, pl.ds(ph * NPH, NPH)], dsts_v)
            gath(0, rows0, sem0).start()
            if P > 0:
                lax.fori_loop(0, P, body, 0)
            if NPH - 2 * P == 1:
                gath(NPH - 1, rows0, sem0).wait()
                scat(NPH - 1, rows0)
            else:
                gath(NPH - 2, rows0, sem0).wait()
                gath(NPH - 1, rows1, sem1).start()
                scat(NPH - 2, rows0)
                gath(NPH - 1, rows1, sem1).wait()
                scat(NPH - 1, rows1)
        plsc.subcore_barrier()
        for t in range(RT // SZ):
            pltpu.sync_copy(acc.at[pl.ds(s * RT + t * SZ, SZ)],
                            rows0.at[pl.ds(0, SZ)])
            pltpu.sync_copy(rows0.at[pl.ds(0, SZ)],
                            out_hbm.at[c, pl.ds(s * RT + t * SZ, SZ)])

    return seg


def _make_segq_kernel(N, NP, RT, EPT, NCH, G):
    mesh = plsc.VectorSubcoreMesh(core_axis_name="c", subcore_axis_name="s")
    NG = NCH // G

    @functools.partial(
        pl.kernel,
        mesh=mesh,
        out_type=jax.ShapeDtypeStruct((NC * NP,), jnp.float32),
        scratch_types=[
            pltpu.VMEM((NCH, K), jnp.int32),
            pltpu.VMEM((NCH, K), jnp.int32),
            pltpu.VMEM((2, G, K), jnp.float32),
            pltpu.VMEM((SZ,), jnp.float32),
            pltpu.VMEM_SHARED((NP,), jnp.float32),
            pltpu.SemaphoreType.DMA,
            pltpu.SemaphoreType.DMA,
            pltpu.SemaphoreType.DMA,
            pltpu.SemaphoreType.DMA,
        ],
    )
    def segq(p_hbm, src_hbm, dst_hbm, z_hbm, out_hbm,
             srcs_v, dsts_v, vals, stz, qacc, semg0, semg1, sems0, sems1):
        c = lax.axis_index("c")
        s = lax.axis_index("s")
        wid = s * NC + c
        pltpu.sync_copy(z_hbm, stz)
        for t in range(RT // SZ):
            pltpu.sync_copy(stz, qacc.at[pl.ds(s * RT + t * SZ, SZ)])
        pltpu.sync_copy(src_hbm.at[wid], srcs_v)
        pltpu.sync_copy(dst_hbm.at[wid], dsts_v)
        plsc.subcore_barrier()
        semg = (semg0, semg1)
        sems = (sems0, sems1)

        def gath(chunk, par, j):
            return pltpu.make_async_copy(
                p_hbm.at[srcs_v.at[chunk]], vals.at[par, j], semg[par])

        def scat(chunk, par, j):
            return pltpu.make_async_copy(
                vals.at[par, j], qacc.at[dsts_v.at[chunk]], sems[par])

        def fire_gath(g, par):
            for j in range(G):
                gath(g * G + j, par, j).start()

        def drain_gath(g, par):
            for j in range(G):
                gath(g * G + j, par, j).wait()

        def fire_scat(g, par):
            for j in range(G):
                scat(g * G + j, par, j).start(add=True)

        def drain_scat(g, par):
            for j in range(G):
                scat(g * G + j, par, j).wait()

        fire_gath(0, 0)

        def body(t, carry):
            g0 = 2 * t
            drain_gath(g0, 0)
            fire_scat(g0, 0)
            fire_gath(g0 + 1, 1)
            drain_gath(g0 + 1, 1)
            fire_scat(g0 + 1, 1)
            drain_scat(g0, 0)
            fire_gath(g0 + 2, 0)
            drain_scat(g0 + 1, 1)
            return carry

        if NG > 2:
            lax.fori_loop(0, NG // 2 - 1, body, 0)
        g0 = NG - 2
        drain_gath(g0, 0)
        fire_scat(g0, 0)
        fire_gath(g0 + 1, 1)
        drain_gath(g0 + 1, 1)
        fire_scat(g0 + 1, 1)
        drain_scat(g0, 0)
        drain_scat(g0 + 1, 1)
        plsc.subcore_barrier()
        for t in range(RT // SZ):
            pltpu.sync_copy(qacc.at[pl.ds(s * RT + t * SZ, SZ)], stz)
            pltpu.sync_copy(stz, out_hbm.at[pl.ds(c * NP + s * RT + t * SZ, SZ)])

    return segq


def _dot(a, b):
    return jnp.dot(a, b, preferred_element_type=jnp.float32,
                   precision=lax.Precision.HIGHEST)


def _enc_body(x_ref, w_ref, b_ref, o_ref):
    o_ref[...] = _dot(x_ref[...], w_ref[...]) + b_ref[...]


def _mid_body(h_ref, ma_ref, mb_ref, ws_ref, wn_ref, b_ref, wc_ref, o_ref):
    m = ma_ref[0] + mb_ref[0]
    h1 = _dot(h_ref[...], ws_ref[...]) + _dot(m, wn_ref[...])
    h1 = jnp.maximum(h1 + b_ref[...], 0.0)
    o_ref[...] = _dot(h1, wc_ref[...])


@jax.jit
def kernel(x, edge_index, W_enc, b_enc, W_self0, W_nei0, b0,
           W_self1, W_nei1, b1, W_head, b_head, num_seed):
    N, C = x.shape
    E = edge_index.shape[1]
    RT = _rup(-(-N // NS), SZ)
    NP = RT * NS
    EPAD = _rup(E, NW * K * 16)
    if EPAD != E and NP == N:
        RT += SZ
        NP = RT * NS
    EPT = EPAD // NW
    NCH = EPT // K

    src = edge_index[0]
    dst = edge_index[1]
    if EPAD != E:
        src = jnp.concatenate([src, jnp.zeros((EPAD - E,), jnp.int32)])
        dst = jnp.concatenate([dst, jnp.full((EPAD - E,), N, jnp.int32)])
    src = src.reshape(NW, NCH, K)
    dst = dst.reshape(NW, NCH, K)

    BR = 1000 if N % 1000 == 0 else 8
    NB = N // BR
    row_spec = pl.BlockSpec((BR, C), lambda i: (i, 0))
    w_spec = pl.BlockSpec((C, C), lambda i: (0, 0))
    b_spec = pl.BlockSpec((1, C), lambda i: (0, 0))

    h = pl.pallas_call(
        _enc_body,
        grid=(NB,),
        in_specs=[row_spec, w_spec, b_spec],
        out_specs=row_spec,
        out_shape=jax.ShapeDtypeStruct((N, C), jnp.float32),
    )(x, W_enc, b_enc.reshape(1, C))

    zrow = jnp.zeros((SZ, C), jnp.float32)
    seg = _make_seg_kernel(N, C, NP, RT, EPT, NCH)
    msg = seg(h, src, dst, zrow)

    wnh = W_nei1 @ W_head
    wsh = W_self1 @ W_head
    Wc = jnp.concatenate([wnh, wsh], axis=1)
    ps = pl.pallas_call(
        _mid_body,
        grid=(NB,),
        in_specs=[row_spec,
                  pl.BlockSpec((1, BR, C), lambda i: (0, i, 0)),
                  pl.BlockSpec((1, BR, C), lambda i: (1, i, 0)),
                  w_spec, w_spec, b_spec,
                  pl.BlockSpec((C, 2), lambda i: (0, 0))],
        out_specs=pl.BlockSpec((BR, 2), lambda i: (i, 0)),
        out_shape=jax.ShapeDtypeStruct((N, 2), jnp.float32),
    )(h, msg, msg, W_self0, W_nei0, b0.reshape(1, C), Wc)

    p = ps[:, 0] + jnp.float32(0.0)
    s_full = ps[:, 1]
    zq = jnp.zeros((SZ,), jnp.float32)
    segq = _make_segq_kernel(N, NP, RT, EPT, NCH, 8)
    q = segq(p, src, dst, zq).reshape(NC, NP)

    tot = s_full + q[0, :N] + q[1, :N]
    seed = lax.dynamic_slice(tot, (num_seed - 1024,), (1024,))
    return seed[:, None] + (b1 @ W_head)[None, :] + b_head[None, :]
